# baseline (speedup 1.0000x reference)
import jax
import jax.numpy as jnp
from jax import lax
from jax.experimental import pallas as pl
from jax.experimental.pallas import tpu as pltpu

_NUM_CLASSES = 21
_VAR0, _VAR1 = 0.1, 0.2
_THRESHOLD = 0.5
_NEGPOS_RATIO = 3
_NOBJ = 8
_BBLK = 8


def _body(tgt_ref, conf_ref, loc_ref, pri_ref, out_l, out_c, out_n):
    g = pl.program_id(0)
    P = pri_ref.shape[1]
    R = _BBLK

    @pl.when(g == 0)
    def _init():
        out_l[0, 0] = 0.0
        out_c[0, 0] = 0.0
        out_n[0, 0] = 0.0

    pri = pri_ref[...]
    pcx, pcy = pri[0:1, :], pri[1:2, :]
    pw, ph = pri[2:3, :], pri[3:4, :]
    px0 = pcx - pw / 2.0
    py0 = pcy - ph / 2.0
    px1 = pcx + pw / 2.0
    py1 = pcy + ph / 2.0
    area_b = (px1 - px0) * (py1 - py0)

    iota = lax.broadcasted_iota(jnp.int32, (R, P), 1)
    tgt = tgt_ref[...]

    ov_rows = []
    tr = []
    for t in range(_NOBJ):
        ax0 = tgt[:, 5 * t + 0:5 * t + 1]
        ay0 = tgt[:, 5 * t + 1:5 * t + 2]
        ax1 = tgt[:, 5 * t + 2:5 * t + 3]
        ay1 = tgt[:, 5 * t + 3:5 * t + 4]
        lab = tgt[:, 5 * t + 4:5 * t + 5]
        iw = jnp.clip(jnp.minimum(ax1, px1) - jnp.maximum(ax0, px0), 0.0, None)
        ih = jnp.clip(jnp.minimum(ay1, py1) - jnp.maximum(ay0, py0), 0.0, None)
        inter = iw * ih
        area_a = (ax1 - ax0) * (ay1 - ay0)
        ov_rows.append(inter / (area_a + area_b - inter))
        tr.append((ax0, ay0, ax1, ay1, lab))

    btv = ov_rows[0]
    bti = jnp.zeros((R, P), jnp.int32)
    for t in range(1, _NOBJ):
        upd = ov_rows[t] > btv
        bti = jnp.where(upd, t, bti)
        btv = jnp.where(upd, ov_rows[t], btv)

    for t in range(_NOBJ):
        m_t = jnp.max(ov_rows[t], axis=1, keepdims=True)
        p_t = jnp.min(jnp.where(ov_rows[t] == m_t, iota, P), axis=1,
                      keepdims=True)
        mask = iota == p_t
        btv = jnp.where(mask, 2.0, btv)
        bti = jnp.where(mask, t, bti)

    conf = jnp.zeros((R, P), jnp.int32)
    mx0 = jnp.zeros((R, P), jnp.float32)
    my0 = jnp.zeros((R, P), jnp.float32)
    mx1 = jnp.zeros((R, P), jnp.float32)
    my1 = jnp.zeros((R, P), jnp.float32)
    for t in range(_NOBJ):
        sel = bti == t
        ax0, ay0, ax1, ay1, lab = tr[t]
        conf = jnp.where(sel, lab.astype(jnp.int32) + 1, conf)
        mx0 = jnp.where(sel, ax0, mx0)
        my0 = jnp.where(sel, ay0, my0)
        mx1 = jnp.where(sel, ax1, mx1)
        my1 = jnp.where(sel, ay1, my1)
    conf = jnp.where(btv < _THRESHOLD, 0, conf)
    pos = conf > 0
    posf = pos.astype(jnp.float32)

    g_cx = ((mx0 + mx1) / 2.0 - pcx) / (_VAR0 * pw)
    g_cy = ((my0 + my1) / 2.0 - pcy) / (_VAR0 * ph)
    g_w = jnp.log((mx1 - mx0) / pw) / _VAR1
    g_h = jnp.log((my1 - my0) / ph) / _VAR1

    ll = jnp.float32(0.0)
    for i, enc in enumerate((g_cx, g_cy, g_w, g_h)):
        d = loc_ref[i] - enc
        ad = jnp.abs(d)
        sm = jnp.where(ad < 1.0, 0.5 * d * d, ad - 0.5)
        ll = ll + jnp.sum(sm * posf)
    out_l[0, 0] += ll

    c3 = conf_ref[...]
    m3 = jnp.max(c3, axis=1, keepdims=True)
    lse = m3[:, 0, :] + jnp.log(jnp.sum(jnp.exp(c3 - m3), axis=1))
    ci3 = lax.broadcasted_iota(jnp.int32, (R, _NUM_CLASSES, P), 1)
    gathered = jnp.sum(
        jnp.where(ci3 == conf.reshape(R, 1, P), c3, 0.0), axis=1)
    ce = lse - gathered

    num_pos = jnp.sum(pos.astype(jnp.int32), axis=1, keepdims=True)
    out_n[0, 0] += jnp.sum(num_pos).astype(jnp.float32)

    mined = jnp.where(pos, 0.0, ce)
    kk = jnp.minimum(_NEGPOS_RATIO * num_pos, P - 1)
    ans = jnp.zeros(kk.shape, jnp.int32)
    for bit in range(30, -1, -1):
        cand = ans | (1 << bit)
        cand_f = lax.bitcast_convert_type(cand, jnp.float32)
        cnt = jnp.sum((mined >= cand_f).astype(jnp.int32), axis=1,
                      keepdims=True)
        ans = jnp.where(cnt >= kk, cand, ans)
    thr = lax.bitcast_convert_type(ans, jnp.float32)
    gt = mined > thr
    cnt_gt = jnp.sum(gt.astype(jnp.int32), axis=1, keepdims=True)
    sum_gt = jnp.sum(jnp.where(gt, mined, 0.0), axis=1, keepdims=True)
    neg = sum_gt + (kk - cnt_gt).astype(jnp.float32) * thr
    neg = jnp.where(kk > 0, neg, 0.0)
    out_c[0, 0] += jnp.sum(ce * posf) + jnp.sum(neg)


_NCHUNKS = 4


def kernel(loc_data, conf_data, priors, targets):
    B, P, _ = loc_data.shape
    pri_t = jnp.transpose(priors, (1, 0))
    tgt_f = targets.reshape(B, _NOBJ * 5)

    cb = B // _NCHUNKS
    parts = []
    for c in range(_NCHUNKS):
        sl = slice(c * cb, (c + 1) * cb)
        conf_t = jnp.transpose(conf_data[sl], (0, 2, 1))
        loc_t = jnp.transpose(loc_data[sl], (2, 0, 1))
        parts.append(pl.pallas_call(
            _body,
            grid=(cb // _BBLK,),
            in_specs=[
                pl.BlockSpec((_BBLK, _NOBJ * 5), lambda g: (g, 0)),
                pl.BlockSpec((_BBLK, _NUM_CLASSES, P), lambda g: (g, 0, 0)),
                pl.BlockSpec((4, _BBLK, P), lambda g: (0, g, 0)),
                pl.BlockSpec((4, P), lambda g: (0, 0)),
            ],
            out_specs=[pl.BlockSpec((1, 1), lambda g: (0, 0),
                                    memory_space=pltpu.SMEM)] * 3,
            out_shape=[jax.ShapeDtypeStruct((1, 1), jnp.float32)] * 3,
        )(tgt_f[sl], conf_t, loc_t, pri_t))
    out_l = sum(p[0][0, 0] for p in parts)
    out_c = sum(p[1][0, 0] for p in parts)
    n = sum(p[2][0, 0] for p in parts)
    return (out_l / n, out_c / n)


# speedup vs baseline: 1.1808x; 1.1808x over previous
import jax
import jax.numpy as jnp
from jax import lax
from jax.experimental import pallas as pl
from jax.experimental.pallas import tpu as pltpu

_NUM_CLASSES = 21
_VAR0, _VAR1 = 0.1, 0.2
_THRESHOLD = 0.5
_NEGPOS_RATIO = 3
_NOBJ = 8
_BBLK = 8


def _body(tgt_ref, conf_ref, loc_ref, pri_ref, out_l, out_c, out_n):
    g = pl.program_id(0)
    P = pri_ref.shape[1]
    R = _BBLK

    @pl.when(g == 0)
    def _init():
        out_l[0, 0] = 0.0
        out_c[0, 0] = 0.0
        out_n[0, 0] = 0.0

    pri = pri_ref[...]
    pcx, pcy = pri[0:1, :], pri[1:2, :]
    pw, ph = pri[2:3, :], pri[3:4, :]
    px0 = pcx - pw / 2.0
    py0 = pcy - ph / 2.0
    px1 = pcx + pw / 2.0
    py1 = pcy + ph / 2.0
    area_b = (px1 - px0) * (py1 - py0)

    iota = lax.broadcasted_iota(jnp.int32, (R, P), 1)
    tgt = tgt_ref[...]

    ov_rows = []
    tr = []
    for t in range(_NOBJ):
        ax0 = tgt[:, 5 * t + 0:5 * t + 1]
        ay0 = tgt[:, 5 * t + 1:5 * t + 2]
        ax1 = tgt[:, 5 * t + 2:5 * t + 3]
        ay1 = tgt[:, 5 * t + 3:5 * t + 4]
        lab = tgt[:, 5 * t + 4:5 * t + 5]
        iw = jnp.clip(jnp.minimum(ax1, px1) - jnp.maximum(ax0, px0), 0.0, None)
        ih = jnp.clip(jnp.minimum(ay1, py1) - jnp.maximum(ay0, py0), 0.0, None)
        inter = iw * ih
        area_a = (ax1 - ax0) * (ay1 - ay0)
        ov_rows.append(inter / (area_a + area_b - inter))
        tr.append((ax0, ay0, ax1, ay1, lab))

    btv = ov_rows[0]
    bti = jnp.zeros((R, P), jnp.int32)
    for t in range(1, _NOBJ):
        upd = ov_rows[t] > btv
        bti = jnp.where(upd, t, bti)
        btv = jnp.where(upd, ov_rows[t], btv)

    for t in range(_NOBJ):
        m_t = jnp.max(ov_rows[t], axis=1, keepdims=True)
        p_t = jnp.min(jnp.where(ov_rows[t] == m_t, iota, P), axis=1,
                      keepdims=True)
        mask = iota == p_t
        btv = jnp.where(mask, 2.0, btv)
        bti = jnp.where(mask, t, bti)

    conf = jnp.zeros((R, P), jnp.int32)
    mx0 = jnp.zeros((R, P), jnp.float32)
    my0 = jnp.zeros((R, P), jnp.float32)
    mx1 = jnp.zeros((R, P), jnp.float32)
    my1 = jnp.zeros((R, P), jnp.float32)
    for t in range(_NOBJ):
        sel = bti == t
        ax0, ay0, ax1, ay1, lab = tr[t]
        conf = jnp.where(sel, lab.astype(jnp.int32) + 1, conf)
        mx0 = jnp.where(sel, ax0, mx0)
        my0 = jnp.where(sel, ay0, my0)
        mx1 = jnp.where(sel, ax1, mx1)
        my1 = jnp.where(sel, ay1, my1)
    conf = jnp.where(btv < _THRESHOLD, 0, conf)
    pos = conf > 0
    posf = pos.astype(jnp.float32)

    g_cx = ((mx0 + mx1) / 2.0 - pcx) / (_VAR0 * pw)
    g_cy = ((my0 + my1) / 2.0 - pcy) / (_VAR0 * ph)
    g_w = jnp.log((mx1 - mx0) / pw) / _VAR1
    g_h = jnp.log((my1 - my0) / ph) / _VAR1

    ll = jnp.float32(0.0)
    for i, enc in enumerate((g_cx, g_cy, g_w, g_h)):
        d = loc_ref[i] - enc
        ad = jnp.abs(d)
        sm = jnp.where(ad < 1.0, 0.5 * d * d, ad - 0.5)
        ll = ll + jnp.sum(sm * posf)
    out_l[0, 0] += ll

    c3 = conf_ref[...]
    m3 = jnp.max(c3, axis=1, keepdims=True)
    lse = m3[:, 0, :] + jnp.log(jnp.sum(jnp.exp(c3 - m3), axis=1))
    ci3 = lax.broadcasted_iota(jnp.int32, (R, _NUM_CLASSES, P), 1)
    gathered = jnp.sum(
        jnp.where(ci3 == conf.reshape(R, 1, P), c3, 0.0), axis=1)
    ce = lse - gathered

    num_pos = jnp.sum(pos.astype(jnp.int32), axis=1, keepdims=True)
    out_n[0, 0] += jnp.sum(num_pos).astype(jnp.float32)

    mined = jnp.where(pos, 0.0, ce)
    kk = jnp.minimum(_NEGPOS_RATIO * num_pos, P - 1)
    ans = jnp.zeros(kk.shape, jnp.int32)
    for bit in range(30, -1, -1):
        cand = ans | (1 << bit)
        cand_f = lax.bitcast_convert_type(cand, jnp.float32)
        cnt = jnp.sum((mined >= cand_f).astype(jnp.int32), axis=1,
                      keepdims=True)
        ans = jnp.where(cnt >= kk, cand, ans)
    thr = lax.bitcast_convert_type(ans, jnp.float32)
    gt = mined > thr
    cnt_gt = jnp.sum(gt.astype(jnp.int32), axis=1, keepdims=True)
    sum_gt = jnp.sum(jnp.where(gt, mined, 0.0), axis=1, keepdims=True)
    neg = sum_gt + (kk - cnt_gt).astype(jnp.float32) * thr
    neg = jnp.where(kk > 0, neg, 0.0)
    out_c[0, 0] += jnp.sum(ce * posf) + jnp.sum(neg)


_NCHUNKS = 1


def kernel(loc_data, conf_data, priors, targets):
    B, P, _ = loc_data.shape
    pri_t = jnp.transpose(priors, (1, 0))
    tgt_f = targets.reshape(B, _NOBJ * 5)

    cb = B // _NCHUNKS
    parts = []
    for c in range(_NCHUNKS):
        sl = slice(c * cb, (c + 1) * cb)
        conf_t = jnp.transpose(conf_data[sl], (0, 2, 1))
        loc_t = jnp.transpose(loc_data[sl], (2, 0, 1))
        parts.append(pl.pallas_call(
            _body,
            grid=(cb // _BBLK,),
            in_specs=[
                pl.BlockSpec((_BBLK, _NOBJ * 5), lambda g: (g, 0)),
                pl.BlockSpec((_BBLK, _NUM_CLASSES, P), lambda g: (g, 0, 0)),
                pl.BlockSpec((4, _BBLK, P), lambda g: (0, g, 0)),
                pl.BlockSpec((4, P), lambda g: (0, 0)),
            ],
            out_specs=[pl.BlockSpec((1, 1), lambda g: (0, 0),
                                    memory_space=pltpu.SMEM)] * 3,
            out_shape=[jax.ShapeDtypeStruct((1, 1), jnp.float32)] * 3,
        )(tgt_f[sl], conf_t, loc_t, pri_t))
    out_l = sum(p[0][0, 0] for p in parts)
    out_c = sum(p[1][0, 0] for p in parts)
    n = sum(p[2][0, 0] for p in parts)
    return (out_l / n, out_c / n)


# speedup vs baseline: 1.5151x; 1.2831x over previous
import jax
import jax.numpy as jnp
from jax import lax
from jax.experimental import pallas as pl
from jax.experimental.pallas import tpu as pltpu

_NUM_CLASSES = 21
_VAR0, _VAR1 = 0.1, 0.2
_THRESHOLD = 0.5
_NEGPOS_RATIO = 3
_NOBJ = 8
_BBLK = 8


def _match_body(tgt_ref, loc_ref, pri_ref, out_l, out_n, conf_out, k_out):
    g = pl.program_id(0)
    P = pri_ref.shape[2]
    R = _BBLK

    @pl.when(g == 0)
    def _init():
        out_l[0, 0] = 0.0
        out_n[0, 0] = 0.0

    pcx = pri_ref[0]
    pcy = pri_ref[1]
    pw = pri_ref[2]
    ph = pri_ref[3]
    px0 = pcx - pw / 2.0
    py0 = pcy - ph / 2.0
    px1 = pcx + pw / 2.0
    py1 = pcy + ph / 2.0
    area_b = (px1 - px0) * (py1 - py0)

    iota = lax.broadcasted_iota(jnp.int32, (R, P), 1)
    tgt = tgt_ref[...]

    ov_rows = []
    tr = []
    for t in range(_NOBJ):
        ax0 = tgt[:, 5 * t + 0:5 * t + 1]
        ay0 = tgt[:, 5 * t + 1:5 * t + 2]
        ax1 = tgt[:, 5 * t + 2:5 * t + 3]
        ay1 = tgt[:, 5 * t + 3:5 * t + 4]
        lab = tgt[:, 5 * t + 4:5 * t + 5]
        iw = jnp.clip(jnp.minimum(ax1, px1) - jnp.maximum(ax0, px0), 0.0, None)
        ih = jnp.clip(jnp.minimum(ay1, py1) - jnp.maximum(ay0, py0), 0.0, None)
        inter = iw * ih
        area_a = (ax1 - ax0) * (ay1 - ay0)
        ov_rows.append(inter / (area_a + area_b - inter))
        tr.append((ax0, ay0, ax1, ay1, lab))

    btv = ov_rows[0]
    bti = jnp.zeros((R, P), jnp.int32)
    for t in range(1, _NOBJ):
        upd = ov_rows[t] > btv
        bti = jnp.where(upd, t, bti)
        btv = jnp.where(upd, ov_rows[t], btv)

    for t in range(_NOBJ):
        m_t = jnp.max(ov_rows[t], axis=1, keepdims=True)
        p_t = jnp.min(jnp.where(ov_rows[t] == m_t, iota, P), axis=1,
                      keepdims=True)
        mask = iota == p_t
        btv = jnp.where(mask, 2.0, btv)
        bti = jnp.where(mask, t, bti)

    conf = jnp.zeros((R, P), jnp.int32)
    mx0 = jnp.zeros((R, P), jnp.float32)
    my0 = jnp.zeros((R, P), jnp.float32)
    mx1 = jnp.zeros((R, P), jnp.float32)
    my1 = jnp.zeros((R, P), jnp.float32)
    for t in range(_NOBJ):
        sel = bti == t
        ax0, ay0, ax1, ay1, lab = tr[t]
        conf = jnp.where(sel, lab.astype(jnp.int32) + 1, conf)
        mx0 = jnp.where(sel, ax0, mx0)
        my0 = jnp.where(sel, ay0, my0)
        mx1 = jnp.where(sel, ax1, mx1)
        my1 = jnp.where(sel, ay1, my1)
    conf = jnp.where(btv < _THRESHOLD, 0, conf)
    pos = conf > 0
    posf = pos.astype(jnp.float32)
    conf_out[...] = conf

    g_cx = ((mx0 + mx1) / 2.0 - pcx) / (_VAR0 * pw)
    g_cy = ((my0 + my1) / 2.0 - pcy) / (_VAR0 * ph)
    g_w = jnp.log((mx1 - mx0) / pw) / _VAR1
    g_h = jnp.log((my1 - my0) / ph) / _VAR1

    ll = jnp.float32(0.0)
    for i, enc in enumerate((g_cx, g_cy, g_w, g_h)):
        d = loc_ref[i] - enc
        ad = jnp.abs(d)
        sm = jnp.where(ad < 1.0, 0.5 * d * d, ad - 0.5)
        ll = ll + jnp.sum(sm * posf)
    out_l[0, 0] += ll

    num_pos = jnp.sum(pos.astype(jnp.int32), axis=1, keepdims=True)
    out_n[0, 0] += jnp.sum(num_pos).astype(jnp.float32)
    k_out[...] = jnp.minimum(_NEGPOS_RATIO * num_pos, P - 1)


def _ce_body(conf_ref, cls_ref, kall_ref, out_c, mined_scr):
    g = pl.program_id(0)
    ng = pl.num_programs(0)
    P = conf_ref.shape[2]
    R = _BBLK

    @pl.when(g == 0)
    def _init():
        out_c[0, 0] = 0.0

    conf = cls_ref[...]
    pos = conf > 0
    posf = pos.astype(jnp.float32)

    c3 = conf_ref[...]
    m3 = jnp.max(c3, axis=1, keepdims=True)
    lse = m3[:, 0, :] + jnp.log(jnp.sum(jnp.exp(c3 - m3), axis=1))
    ci3 = lax.broadcasted_iota(jnp.int32, (R, _NUM_CLASSES, P), 1)
    gathered = jnp.sum(
        jnp.where(ci3 == conf.reshape(R, 1, P), c3, 0.0), axis=1)
    ce = lse - gathered

    out_c[0, 0] += jnp.sum(ce * posf)
    mined_scr[pl.ds(g * R, R), :] = jnp.where(pos, 0.0, ce)

    @pl.when(g == ng - 1)
    def _mine():
        mm = mined_scr[...]
        kk = kall_ref[...]
        ans = jnp.zeros(kk.shape, jnp.int32)
        for bit in range(30, -1, -1):
            cand = ans | (1 << bit)
            cand_f = lax.bitcast_convert_type(cand, jnp.float32)
            cnt = jnp.sum((mm >= cand_f).astype(jnp.int32), axis=1,
                          keepdims=True)
            ans = jnp.where(cnt >= kk, cand, ans)
        thr = lax.bitcast_convert_type(ans, jnp.float32)
        gt = mm > thr
        cnt_gt = jnp.sum(gt.astype(jnp.int32), axis=1, keepdims=True)
        sum_gt = jnp.sum(jnp.where(gt, mm, 0.0), axis=1, keepdims=True)
        neg = sum_gt + (kk - cnt_gt).astype(jnp.float32) * thr
        neg = jnp.where(kk > 0, neg, 0.0)
        out_c[0, 0] += jnp.sum(neg)


def kernel(loc_data, conf_data, priors, targets):
    B, P, _ = loc_data.shape
    conf_t = jnp.transpose(conf_data, (0, 2, 1))
    loc_t = jnp.transpose(loc_data, (2, 0, 1))
    pri_t = jnp.broadcast_to(
        jnp.transpose(priors, (1, 0))[:, None, :], (4, _BBLK, P))
    tgt_f = targets.reshape(B, _NOBJ * 5)

    out_l, out_n, cls_t, k_all = pl.pallas_call(
        _match_body,
        grid=(B // _BBLK,),
        in_specs=[
            pl.BlockSpec((_BBLK, _NOBJ * 5), lambda g: (g, 0)),
            pl.BlockSpec((4, _BBLK, P), lambda g: (0, g, 0)),
            pl.BlockSpec((4, _BBLK, P), lambda g: (0, 0, 0)),
        ],
        out_specs=[
            pl.BlockSpec((1, 1), lambda g: (0, 0), memory_space=pltpu.SMEM),
            pl.BlockSpec((1, 1), lambda g: (0, 0), memory_space=pltpu.SMEM),
            pl.BlockSpec((_BBLK, P), lambda g: (g, 0)),
            pl.BlockSpec((_BBLK, 1), lambda g: (g, 0)),
        ],
        out_shape=[
            jax.ShapeDtypeStruct((1, 1), jnp.float32),
            jax.ShapeDtypeStruct((1, 1), jnp.float32),
            jax.ShapeDtypeStruct((B, P), jnp.int32),
            jax.ShapeDtypeStruct((B, 1), jnp.int32),
        ],
    )(tgt_f, loc_t, pri_t)

    out_c, = pl.pallas_call(
        _ce_body,
        grid=(B // _BBLK,),
        in_specs=[
            pl.BlockSpec((_BBLK, _NUM_CLASSES, P), lambda g: (g, 0, 0)),
            pl.BlockSpec((_BBLK, P), lambda g: (g, 0)),
            pl.BlockSpec((B, 1), lambda g: (0, 0)),
        ],
        out_specs=[
            pl.BlockSpec((1, 1), lambda g: (0, 0), memory_space=pltpu.SMEM),
        ],
        out_shape=[jax.ShapeDtypeStruct((1, 1), jnp.float32)],
        scratch_shapes=[pltpu.VMEM((B, P), jnp.float32)],
    )(conf_t, cls_t, k_all)

    n = out_n[0, 0]
    return (out_l[0, 0] / n, out_c[0, 0] / n)


# speedup vs baseline: 1.5491x; 1.0225x over previous
import jax
import jax.numpy as jnp
from jax import lax
from jax.experimental import pallas as pl
from jax.experimental.pallas import tpu as pltpu

_NUM_CLASSES = 21
_VAR0, _VAR1 = 0.1, 0.2
_THRESHOLD = 0.5
_NEGPOS_RATIO = 3
_NOBJ = 8
_BBLK = 8


def _match_body(tgt_ref, loc_ref, pri_ref, out_l, out_n, conf_out, k_out):
    g = pl.program_id(0)
    P = pri_ref.shape[2]
    R = _BBLK

    @pl.when(g == 0)
    def _init():
        out_l[0, 0] = 0.0
        out_n[0, 0] = 0.0

    pcx = pri_ref[0]
    pcy = pri_ref[1]
    pw = pri_ref[2]
    ph = pri_ref[3]
    px0 = pcx - pw / 2.0
    py0 = pcy - ph / 2.0
    px1 = pcx + pw / 2.0
    py1 = pcy + ph / 2.0
    area_b = (px1 - px0) * (py1 - py0)

    iota = lax.broadcasted_iota(jnp.int32, (R, P), 1)
    tgt = tgt_ref[...]

    ov_rows = []
    tr = []
    for t in range(_NOBJ):
        ax0 = tgt[:, 5 * t + 0:5 * t + 1]
        ay0 = tgt[:, 5 * t + 1:5 * t + 2]
        ax1 = tgt[:, 5 * t + 2:5 * t + 3]
        ay1 = tgt[:, 5 * t + 3:5 * t + 4]
        lab = tgt[:, 5 * t + 4:5 * t + 5]
        iw = jnp.clip(jnp.minimum(ax1, px1) - jnp.maximum(ax0, px0), 0.0, None)
        ih = jnp.clip(jnp.minimum(ay1, py1) - jnp.maximum(ay0, py0), 0.0, None)
        inter = iw * ih
        area_a = (ax1 - ax0) * (ay1 - ay0)
        ov_rows.append(inter / (area_a + area_b - inter))
        tr.append((ax0, ay0, ax1, ay1, lab))

    btv = ov_rows[0]
    bti = jnp.zeros((R, P), jnp.int32)
    for t in range(1, _NOBJ):
        upd = ov_rows[t] > btv
        bti = jnp.where(upd, t, bti)
        btv = jnp.where(upd, ov_rows[t], btv)

    for t in range(_NOBJ):
        m_t = jnp.max(ov_rows[t], axis=1, keepdims=True)
        p_t = jnp.min(jnp.where(ov_rows[t] == m_t, iota, P), axis=1,
                      keepdims=True)
        mask = iota == p_t
        btv = jnp.where(mask, 2.0, btv)
        bti = jnp.where(mask, t, bti)

    conf = jnp.zeros((R, P), jnp.int32)
    mx0 = jnp.zeros((R, P), jnp.float32)
    my0 = jnp.zeros((R, P), jnp.float32)
    mx1 = jnp.zeros((R, P), jnp.float32)
    my1 = jnp.zeros((R, P), jnp.float32)
    for t in range(_NOBJ):
        sel = bti == t
        ax0, ay0, ax1, ay1, lab = tr[t]
        conf = jnp.where(sel, lab.astype(jnp.int32) + 1, conf)
        mx0 = jnp.where(sel, ax0, mx0)
        my0 = jnp.where(sel, ay0, my0)
        mx1 = jnp.where(sel, ax1, mx1)
        my1 = jnp.where(sel, ay1, my1)
    conf = jnp.where(btv < _THRESHOLD, 0, conf)
    pos = conf > 0
    posf = pos.astype(jnp.float32)
    conf_out[...] = conf

    g_cx = ((mx0 + mx1) / 2.0 - pcx) / (_VAR0 * pw)
    g_cy = ((my0 + my1) / 2.0 - pcy) / (_VAR0 * ph)
    g_w = jnp.log((mx1 - mx0) / pw) / _VAR1
    g_h = jnp.log((my1 - my0) / ph) / _VAR1

    ll = jnp.float32(0.0)
    for i, enc in enumerate((g_cx, g_cy, g_w, g_h)):
        d = loc_ref[:, i, :] - enc
        ad = jnp.abs(d)
        sm = jnp.where(ad < 1.0, 0.5 * d * d, ad - 0.5)
        ll = ll + jnp.sum(sm * posf)
    out_l[0, 0] += ll

    num_pos = jnp.sum(pos.astype(jnp.int32), axis=1, keepdims=True)
    out_n[0, 0] += jnp.sum(num_pos).astype(jnp.float32)
    k_out[...] = jnp.minimum(_NEGPOS_RATIO * num_pos, P - 1)


def _ce_body(conf_ref, cls_ref, kall_ref, out_c, mined_scr):
    g = pl.program_id(0)
    ng = pl.num_programs(0)
    P = conf_ref.shape[2]
    R = _BBLK

    @pl.when(g == 0)
    def _init():
        out_c[0, 0] = 0.0

    conf = cls_ref[...]
    pos = conf > 0
    posf = pos.astype(jnp.float32)

    c3 = conf_ref[...]
    m3 = jnp.max(c3, axis=1, keepdims=True)
    lse = m3[:, 0, :] + jnp.log(jnp.sum(jnp.exp(c3 - m3), axis=1))
    ci3 = lax.broadcasted_iota(jnp.int32, (R, _NUM_CLASSES, P), 1)
    gathered = jnp.sum(
        jnp.where(ci3 == conf.reshape(R, 1, P), c3, 0.0), axis=1)
    ce = lse - gathered

    out_c[0, 0] += jnp.sum(ce * posf)
    mined_scr[pl.ds(g * R, R), :] = jnp.where(pos, 0.0, ce)

    @pl.when(g == ng - 1)
    def _mine():
        mm = mined_scr[...]
        kk = kall_ref[...]
        ans = jnp.zeros(kk.shape, jnp.int32)
        for bit in range(30, -1, -1):
            cand = ans | (1 << bit)
            cand_f = lax.bitcast_convert_type(cand, jnp.float32)
            cnt = jnp.sum((mm >= cand_f).astype(jnp.int32), axis=1,
                          keepdims=True)
            ans = jnp.where(cnt >= kk, cand, ans)
        thr = lax.bitcast_convert_type(ans, jnp.float32)
        gt = mm > thr
        cnt_gt = jnp.sum(gt.astype(jnp.int32), axis=1, keepdims=True)
        sum_gt = jnp.sum(jnp.where(gt, mm, 0.0), axis=1, keepdims=True)
        neg = sum_gt + (kk - cnt_gt).astype(jnp.float32) * thr
        neg = jnp.where(kk > 0, neg, 0.0)
        out_c[0, 0] += jnp.sum(neg)


def kernel(loc_data, conf_data, priors, targets):
    B, P, _ = loc_data.shape
    conf_t = jnp.transpose(conf_data, (0, 2, 1))
    loc_t = jnp.transpose(loc_data, (0, 2, 1))
    pri_t = jnp.broadcast_to(
        jnp.transpose(priors, (1, 0))[:, None, :], (4, _BBLK, P))
    tgt_f = targets.reshape(B, _NOBJ * 5)

    out_l, out_n, cls_t, k_all = pl.pallas_call(
        _match_body,
        grid=(B // _BBLK,),
        in_specs=[
            pl.BlockSpec((_BBLK, _NOBJ * 5), lambda g: (g, 0)),
            pl.BlockSpec((_BBLK, 4, P), lambda g: (g, 0, 0)),
            pl.BlockSpec((4, _BBLK, P), lambda g: (0, 0, 0)),
        ],
        out_specs=[
            pl.BlockSpec((1, 1), lambda g: (0, 0), memory_space=pltpu.SMEM),
            pl.BlockSpec((1, 1), lambda g: (0, 0), memory_space=pltpu.SMEM),
            pl.BlockSpec((_BBLK, P), lambda g: (g, 0)),
            pl.BlockSpec((_BBLK, 1), lambda g: (g, 0)),
        ],
        out_shape=[
            jax.ShapeDtypeStruct((1, 1), jnp.float32),
            jax.ShapeDtypeStruct((1, 1), jnp.float32),
            jax.ShapeDtypeStruct((B, P), jnp.int32),
            jax.ShapeDtypeStruct((B, 1), jnp.int32),
        ],
    )(tgt_f, loc_t, pri_t)

    out_c, = pl.pallas_call(
        _ce_body,
        grid=(B // _BBLK,),
        in_specs=[
            pl.BlockSpec((_BBLK, _NUM_CLASSES, P), lambda g: (g, 0, 0)),
            pl.BlockSpec((_BBLK, P), lambda g: (g, 0)),
            pl.BlockSpec((B, 1), lambda g: (0, 0)),
        ],
        out_specs=[
            pl.BlockSpec((1, 1), lambda g: (0, 0), memory_space=pltpu.SMEM),
        ],
        out_shape=[jax.ShapeDtypeStruct((1, 1), jnp.float32)],
        scratch_shapes=[pltpu.VMEM((B, P), jnp.float32)],
    )(conf_t, cls_t, k_all)

    n = out_n[0, 0]
    return (out_l[0, 0] / n, out_c[0, 0] / n)
